# Initial kernel scaffold; baseline (speedup 1.0000x reference)
import jax
import jax.numpy as jnp
from jax.experimental import pallas as pl


def kernel(x, edge_index, W1a, b1a, W1b, b1b, eps1, g1, be1, W2a, b2a, W2b, b2b, eps2, g2, be2, Wf, bf):
    raise NotImplementedError("write your pallas kernel here")


# speedup vs baseline: 21.0669x; 21.0669x over previous
import functools

import jax
import jax.numpy as jnp
from jax import lax
from jax.experimental import pallas as pl
from jax.experimental.pallas import tpu as pltpu
from jax.experimental.pallas import tpu_sc as plsc

_N = 10000
_E = 320000
_D_IN = 128
_DIM = 16
_N_CLASSES = 40

_NC = 2
_NS = 16
_NW = _NC * _NS
_EW = _E // _NW
_CHUNK = 2000
_NCHUNK = _EW // _CHUNK
_NPAD = 10240
_ROWS = _NPAD // _NS


def _make_segsum():
    mesh = plsc.VectorSubcoreMesh(core_axis_name="c", subcore_axis_name="s")

    @functools.partial(
        pl.kernel,
        mesh=mesh,
        compiler_params=pltpu.CompilerParams(use_tc_tiling_on_sc=False),
        out_type=jax.ShapeDtypeStruct((_NC, _NPAD, _DIM), jnp.float32),
        scratch_types=[
            pltpu.VMEM((_CHUNK,), jnp.int32),
            pltpu.VMEM((_CHUNK,), jnp.int32),
            pltpu.VMEM((_CHUNK, _DIM), jnp.float32),
            pltpu.VMEM((_ROWS, _DIM), jnp.float32),
            pltpu.VMEM_SHARED((_NPAD, _DIM), jnp.float32),
            pltpu.SemaphoreType.DMA,
        ],
    )
    def segsum(p_hbm, src_hbm, dst_hbm, out_hbm,
               src_v, dst_v, rows_v, zbuf_v, acc_s, sem):
        cid = lax.axis_index("c")
        sid = lax.axis_index("s")
        wid = cid * _NS + sid

        def zrow(i, carry):
            zbuf_v[i, :] = jnp.zeros((_DIM,), jnp.float32)
            return carry

        lax.fori_loop(0, _ROWS, zrow, 0)
        pltpu.sync_copy(zbuf_v, acc_s.at[pl.ds(sid * _ROWS, _ROWS)])
        plsc.subcore_barrier()

        def chunk(i, carry):
            base = pl.multiple_of(wid * _EW + i * _CHUNK, 8)
            pltpu.sync_copy(src_hbm.at[pl.ds(base, _CHUNK)], src_v)
            pltpu.sync_copy(dst_hbm.at[pl.ds(base, _CHUNK)], dst_v)
            pltpu.async_copy(p_hbm.at[src_v], rows_v, sem).wait()
            pltpu.sync_copy(rows_v, acc_s.at[dst_v], add=True)
            return carry

        lax.fori_loop(0, _NCHUNK, chunk, 0)
        plsc.subcore_barrier()

        pltpu.sync_copy(acc_s.at[pl.ds(sid * _ROWS, _ROWS)],
                        out_hbm.at[cid, pl.ds(sid * _ROWS, _ROWS)])

    return segsum


_SEGSUM_CACHE = []


def _segsum(p, src, dst):
    if not _SEGSUM_CACHE:
        _SEGSUM_CACHE.append(_make_segsum())
    return _SEGSUM_CACHE[0](p, src, dst)


def _proj_body(x_ref, w_ref, o_ref):
    o_ref[...] = jnp.dot(x_ref[...], w_ref[...],
                         preferred_element_type=jnp.float32)


def _project(x, w):
    return pl.pallas_call(
        _proj_body,
        out_shape=jax.ShapeDtypeStruct((_N, _DIM), jnp.float32),
    )(x, w)


def _bn(u, g_ref, be_ref):
    mean = jnp.mean(u, axis=0, keepdims=True)
    var = jnp.mean((u - mean) ** 2, axis=0, keepdims=True)
    return g_ref[...] * (u - mean) * lax.rsqrt(var + 1e-5) + be_ref[...]


def _mid_body(p_ref, acc_ref, sc_ref, w1b_ref, b1a_ref, b1b_ref,
              g1_ref, be1_ref, w2a_ref, q_ref):
    u = (sc_ref[0, 0] * p_ref[...] + acc_ref[0, :_N] + acc_ref[1, :_N]
         + b1a_ref[...])
    u = jnp.maximum(u, 0.0)
    u = jnp.dot(u, w1b_ref[...], preferred_element_type=jnp.float32)
    u = jnp.maximum(u + b1b_ref[...], 0.0)
    h = _bn(u, g1_ref, be1_ref)
    q_ref[...] = jnp.dot(h, w2a_ref[...], preferred_element_type=jnp.float32)


def _mid(p, acc, scale1, w1b, b1a, b1b, g1, be1, w2a):
    return pl.pallas_call(
        _mid_body,
        out_shape=jax.ShapeDtypeStruct((_N, _DIM), jnp.float32),
    )(p, acc, scale1, w1b, b1a, b1b, g1, be1, w2a)


def _final_body(q_ref, acc_ref, sc_ref, w2b_ref, b2a_ref, b2b_ref,
                g2_ref, be2_ref, wf_ref, bf_ref, o_ref):
    u = (sc_ref[0, 0] * q_ref[...] + acc_ref[0, :_N] + acc_ref[1, :_N]
         + b2a_ref[...])
    u = jnp.maximum(u, 0.0)
    u = jnp.dot(u, w2b_ref[...], preferred_element_type=jnp.float32)
    u = jnp.maximum(u + b2b_ref[...], 0.0)
    h = _bn(u, g2_ref, be2_ref)
    logits = jnp.dot(h, wf_ref[...],
                     preferred_element_type=jnp.float32) + bf_ref[...]
    m = jnp.max(logits, axis=-1, keepdims=True)
    lse = m + jnp.log(jnp.sum(jnp.exp(logits - m), axis=-1, keepdims=True))
    o_ref[...] = logits - lse


def _final(q, acc, scale2, w2b, b2a, b2b, g2, be2, wf, bf):
    return pl.pallas_call(
        _final_body,
        out_shape=jax.ShapeDtypeStruct((_N, _N_CLASSES), jnp.float32),
    )(q, acc, scale2, w2b, b2a, b2b, g2, be2, wf, bf)


def kernel(x, edge_index, W1a, b1a, W1b, b1b, eps1, g1, be1,
           W2a, b2a, W2b, b2b, eps2, g2, be2, Wf, bf):
    src = edge_index[0].astype(jnp.int32)
    dst = edge_index[1].astype(jnp.int32)
    scale1 = jnp.reshape(1.0 + eps1, (1, 1))
    scale2 = jnp.reshape(1.0 + eps2, (1, 1))

    p = _project(x, W1a)
    acc1 = _segsum(p, src, dst)
    q = _mid(p, acc1, scale1, W1b, jnp.reshape(b1a, (1, _DIM)),
             jnp.reshape(b1b, (1, _DIM)), jnp.reshape(g1, (1, _DIM)),
             jnp.reshape(be1, (1, _DIM)), W2a)
    acc2 = _segsum(q, src, dst)
    return _final(q, acc2, scale2, W2b, jnp.reshape(b2a, (1, _DIM)),
                  jnp.reshape(b2b, (1, _DIM)), jnp.reshape(g2, (1, _DIM)),
                  jnp.reshape(be2, (1, _DIM)), Wf,
                  jnp.reshape(bf, (1, _N_CLASSES)))


# speedup vs baseline: 24.5032x; 1.1631x over previous
import functools

import jax
import jax.numpy as jnp
from jax import lax
from jax.experimental import pallas as pl
from jax.experimental.pallas import tpu as pltpu
from jax.experimental.pallas import tpu_sc as plsc

_N = 10000
_E = 320000
_D_IN = 128
_DIM = 16
_N_CLASSES = 40

_NC = 2
_NS = 16
_NW = _NC * _NS
_EW = _E // _NW
_CHUNK = 2000
_NCHUNK = _EW // _CHUNK
_NPAD = 10240
_ROWS = _NPAD // _NS


def _make_segsum():
    mesh = plsc.VectorSubcoreMesh(core_axis_name="c", subcore_axis_name="s")

    @functools.partial(
        pl.kernel,
        mesh=mesh,
        compiler_params=pltpu.CompilerParams(use_tc_tiling_on_sc=False),
        out_type=jax.ShapeDtypeStruct((_NC, _NPAD, _DIM), jnp.float32),
        scratch_types=[
            pltpu.VMEM((_EW,), jnp.int32),
            pltpu.VMEM((_NCHUNK, _CHUNK), jnp.int32),
            pltpu.VMEM((2, _CHUNK, _DIM), jnp.float32),
            pltpu.VMEM((_ROWS, _DIM), jnp.float32),
            pltpu.VMEM_SHARED((_NPAD, _DIM), jnp.float32),
            pltpu.SemaphoreType.DMA,
            pltpu.SemaphoreType.DMA,
            pltpu.SemaphoreType.DMA,
            pltpu.SemaphoreType.DMA,
        ],
    )
    def segsum(p_hbm, src_hbm, dst2_hbm, out_hbm,
               src_v, dst_v, rows_v, zbuf_v, acc_s,
               sem_s, sem_d, sem_a, sem_b):
        cid = lax.axis_index("c")
        sid = lax.axis_index("s")
        wid = cid * _NS + sid

        ebase = pl.multiple_of(wid * _EW, 8)
        cp_s = pltpu.async_copy(src_hbm.at[pl.ds(ebase, _EW)], src_v, sem_s)
        cp_d = pltpu.async_copy(dst2_hbm.at[pl.ds(wid * _NCHUNK, _NCHUNK)],
                                dst_v, sem_d)

        def zrow(i, carry):
            zbuf_v[i, :] = jnp.zeros((_DIM,), jnp.float32)
            return carry

        lax.fori_loop(0, _ROWS, zrow, 0)
        pltpu.sync_copy(zbuf_v, acc_s.at[pl.ds(sid * _ROWS, _ROWS)])
        cp_s.wait()
        cp_d.wait()

        sems = (sem_a, sem_b)
        gathers = [None, None]
        for i in range(2):
            gathers[i] = pltpu.async_copy(
                p_hbm.at[src_v.at[pl.ds(i * _CHUNK, _CHUNK)]],
                rows_v.at[i], sems[i])
        plsc.subcore_barrier()
        for i in range(_NCHUNK):
            b = i % 2
            gathers[b].wait()
            pltpu.sync_copy(rows_v.at[b], acc_s.at[dst_v.at[i]], add=True)
            nxt = i + 2
            if nxt < _NCHUNK:
                gathers[b] = pltpu.async_copy(
                    p_hbm.at[src_v.at[pl.ds(nxt * _CHUNK, _CHUNK)]],
                    rows_v.at[b], sems[b])
        plsc.subcore_barrier()

        pltpu.sync_copy(acc_s.at[pl.ds(sid * _ROWS, _ROWS)],
                        out_hbm.at[cid, pl.ds(sid * _ROWS, _ROWS)])

    return segsum


_SEGSUM_CACHE = []


def _segsum(p, src, dst2):
    if not _SEGSUM_CACHE:
        _SEGSUM_CACHE.append(_make_segsum())
    return _SEGSUM_CACHE[0](p, src, dst2)


def _proj_body(x_ref, w_ref, o_ref):
    o_ref[...] = jnp.dot(x_ref[...], w_ref[...],
                         preferred_element_type=jnp.float32)


def _project(x, w):
    return pl.pallas_call(
        _proj_body,
        out_shape=jax.ShapeDtypeStruct((_N, _DIM), jnp.float32),
    )(x, w)


def _bn(u, g_ref, be_ref):
    mean = jnp.mean(u, axis=0, keepdims=True)
    var = jnp.mean((u - mean) ** 2, axis=0, keepdims=True)
    return g_ref[...] * (u - mean) * lax.rsqrt(var + 1e-5) + be_ref[...]


def _mid_body(p_ref, acc_ref, sc_ref, w1b_ref, b1a_ref, b1b_ref,
              g1_ref, be1_ref, w2a_ref, q_ref):
    u = (sc_ref[0, 0] * p_ref[...] + acc_ref[0, :_N] + acc_ref[1, :_N]
         + b1a_ref[...])
    u = jnp.maximum(u, 0.0)
    u = jnp.dot(u, w1b_ref[...], preferred_element_type=jnp.float32)
    u = jnp.maximum(u + b1b_ref[...], 0.0)
    h = _bn(u, g1_ref, be1_ref)
    q_ref[...] = jnp.dot(h, w2a_ref[...], preferred_element_type=jnp.float32)


def _mid(p, acc, scale1, w1b, b1a, b1b, g1, be1, w2a):
    return pl.pallas_call(
        _mid_body,
        out_shape=jax.ShapeDtypeStruct((_N, _DIM), jnp.float32),
    )(p, acc, scale1, w1b, b1a, b1b, g1, be1, w2a)


def _final_body(q_ref, acc_ref, sc_ref, w2b_ref, b2a_ref, b2b_ref,
                g2_ref, be2_ref, wf_ref, bf_ref, o_ref):
    u = (sc_ref[0, 0] * q_ref[...] + acc_ref[0, :_N] + acc_ref[1, :_N]
         + b2a_ref[...])
    u = jnp.maximum(u, 0.0)
    u = jnp.dot(u, w2b_ref[...], preferred_element_type=jnp.float32)
    u = jnp.maximum(u + b2b_ref[...], 0.0)
    h = _bn(u, g2_ref, be2_ref)
    logits = jnp.dot(h, wf_ref[...],
                     preferred_element_type=jnp.float32) + bf_ref[...]
    m = jnp.max(logits, axis=-1, keepdims=True)
    lse = m + jnp.log(jnp.sum(jnp.exp(logits - m), axis=-1, keepdims=True))
    o_ref[...] = logits - lse


def _final(q, acc, scale2, w2b, b2a, b2b, g2, be2, wf, bf):
    return pl.pallas_call(
        _final_body,
        out_shape=jax.ShapeDtypeStruct((_N, _N_CLASSES), jnp.float32),
    )(q, acc, scale2, w2b, b2a, b2b, g2, be2, wf, bf)


def kernel(x, edge_index, W1a, b1a, W1b, b1b, eps1, g1, be1,
           W2a, b2a, W2b, b2b, eps2, g2, be2, Wf, bf):
    src = edge_index[0].astype(jnp.int32)
    dst2 = jnp.reshape(edge_index[1].astype(jnp.int32),
                       (_E // _CHUNK, _CHUNK))
    scale1 = jnp.reshape(1.0 + eps1, (1, 1))
    scale2 = jnp.reshape(1.0 + eps2, (1, 1))

    p = _project(x, W1a)
    acc1 = _segsum(p, src, dst2)
    q = _mid(p, acc1, scale1, W1b, jnp.reshape(b1a, (1, _DIM)),
             jnp.reshape(b1b, (1, _DIM)), jnp.reshape(g1, (1, _DIM)),
             jnp.reshape(be1, (1, _DIM)), W2a)
    acc2 = _segsum(q, src, dst2)
    return _final(q, acc2, scale2, W2b, jnp.reshape(b2a, (1, _DIM)),
                  jnp.reshape(b2b, (1, _DIM)), jnp.reshape(g2, (1, _DIM)),
                  jnp.reshape(be2, (1, _DIM)), Wf,
                  jnp.reshape(bf, (1, _N_CLASSES)))


# speedup vs baseline: 25.3856x; 1.0360x over previous
import functools

import jax
import jax.numpy as jnp
from jax import lax
from jax.experimental import pallas as pl
from jax.experimental.pallas import tpu as pltpu
from jax.experimental.pallas import tpu_sc as plsc

_N = 10000
_E = 320000
_D_IN = 128
_DIM = 16
_N_CLASSES = 40

_NC = 2
_NS = 16
_NW = _NC * _NS
_EW = _E // _NW
_CHUNK = 2000
_NCHUNK = _EW // _CHUNK
_NPAD = 10240
_ROWS = _NPAD // _NS


def _make_segsum():
    mesh = plsc.VectorSubcoreMesh(core_axis_name="c", subcore_axis_name="s")

    @functools.partial(
        pl.kernel,
        mesh=mesh,
        compiler_params=pltpu.CompilerParams(use_tc_tiling_on_sc=False),
        out_type=jax.ShapeDtypeStruct((_NC, _NPAD, _DIM), jnp.float32),
        scratch_types=[
            pltpu.VMEM((_EW,), jnp.int32),
            pltpu.VMEM((_NCHUNK, _CHUNK), jnp.int32),
            pltpu.VMEM((2, _CHUNK, _DIM), jnp.float32),
            pltpu.VMEM((_ROWS, _DIM), jnp.float32),
            pltpu.VMEM_SHARED((_NPAD, _DIM), jnp.float32),
            pltpu.VMEM_SHARED((_N, _DIM), jnp.float32),
            pltpu.SemaphoreType.DMA,
            pltpu.SemaphoreType.DMA,
            pltpu.SemaphoreType.DMA,
            pltpu.SemaphoreType.DMA,
        ],
    )
    def segsum(p_hbm, src_hbm, dst2_hbm, out_hbm,
               src_v, dst_v, rows_v, zbuf_v, acc_s, p_s,
               sem_s, sem_d, sem_a, sem_b):
        cid = lax.axis_index("c")
        sid = lax.axis_index("s")
        wid = cid * _NS + sid

        ebase = pl.multiple_of(wid * _EW, 8)
        cp_s = pltpu.async_copy(src_hbm.at[pl.ds(ebase, _EW)], src_v, sem_s)
        cp_d = pltpu.async_copy(dst2_hbm.at[pl.ds(wid * _NCHUNK, _NCHUNK)],
                                dst_v, sem_d)

        @pl.when(sid < _NS - 1)
        def _():
            pltpu.sync_copy(p_hbm.at[pl.ds(sid * 640, 640)],
                            p_s.at[pl.ds(sid * 640, 640)])

        @pl.when(sid == _NS - 1)
        def _():
            pltpu.sync_copy(p_hbm.at[pl.ds(9600, _N - 9600)],
                            p_s.at[pl.ds(9600, _N - 9600)])

        def zrow(i, carry):
            zbuf_v[i, :] = jnp.zeros((_DIM,), jnp.float32)
            return carry

        lax.fori_loop(0, _ROWS, zrow, 0)
        pltpu.sync_copy(zbuf_v, acc_s.at[pl.ds(sid * _ROWS, _ROWS)])
        cp_s.wait()
        cp_d.wait()

        plsc.subcore_barrier()
        sems = (sem_a, sem_b)
        gathers = [None, None]
        for i in range(2):
            gathers[i] = pltpu.async_copy(
                p_s.at[src_v.at[pl.ds(i * _CHUNK, _CHUNK)]],
                rows_v.at[i], sems[i])
        for i in range(_NCHUNK):
            b = i % 2
            gathers[b].wait()
            pltpu.sync_copy(rows_v.at[b], acc_s.at[dst_v.at[i]], add=True)
            nxt = i + 2
            if nxt < _NCHUNK:
                gathers[b] = pltpu.async_copy(
                    p_s.at[src_v.at[pl.ds(nxt * _CHUNK, _CHUNK)]],
                    rows_v.at[b], sems[b])
        plsc.subcore_barrier()

        pltpu.sync_copy(acc_s.at[pl.ds(sid * _ROWS, _ROWS)],
                        out_hbm.at[cid, pl.ds(sid * _ROWS, _ROWS)])

    return segsum


_SEGSUM_CACHE = []


def _segsum(p, src, dst2):
    if not _SEGSUM_CACHE:
        _SEGSUM_CACHE.append(_make_segsum())
    return _SEGSUM_CACHE[0](p, src, dst2)


def _proj_body(x_ref, w_ref, o_ref):
    o_ref[...] = jnp.dot(x_ref[...], w_ref[...],
                         preferred_element_type=jnp.float32)


def _project(x, w):
    return pl.pallas_call(
        _proj_body,
        out_shape=jax.ShapeDtypeStruct((_N, _DIM), jnp.float32),
    )(x, w)


def _bn(u, g_ref, be_ref):
    mean = jnp.mean(u, axis=0, keepdims=True)
    var = jnp.mean((u - mean) ** 2, axis=0, keepdims=True)
    return g_ref[...] * (u - mean) * lax.rsqrt(var + 1e-5) + be_ref[...]


def _mid_body(p_ref, acc_ref, sc_ref, w1b_ref, b1a_ref, b1b_ref,
              g1_ref, be1_ref, w2a_ref, q_ref):
    u = (sc_ref[0, 0] * p_ref[...] + acc_ref[0, :_N] + acc_ref[1, :_N]
         + b1a_ref[...])
    u = jnp.maximum(u, 0.0)
    u = jnp.dot(u, w1b_ref[...], preferred_element_type=jnp.float32)
    u = jnp.maximum(u + b1b_ref[...], 0.0)
    h = _bn(u, g1_ref, be1_ref)
    q_ref[...] = jnp.dot(h, w2a_ref[...], preferred_element_type=jnp.float32)


def _mid(p, acc, scale1, w1b, b1a, b1b, g1, be1, w2a):
    return pl.pallas_call(
        _mid_body,
        out_shape=jax.ShapeDtypeStruct((_N, _DIM), jnp.float32),
    )(p, acc, scale1, w1b, b1a, b1b, g1, be1, w2a)


def _final_body(q_ref, acc_ref, sc_ref, w2b_ref, b2a_ref, b2b_ref,
                g2_ref, be2_ref, wf_ref, bf_ref, o_ref):
    u = (sc_ref[0, 0] * q_ref[...] + acc_ref[0, :_N] + acc_ref[1, :_N]
         + b2a_ref[...])
    u = jnp.maximum(u, 0.0)
    u = jnp.dot(u, w2b_ref[...], preferred_element_type=jnp.float32)
    u = jnp.maximum(u + b2b_ref[...], 0.0)
    h = _bn(u, g2_ref, be2_ref)
    logits = jnp.dot(h, wf_ref[...],
                     preferred_element_type=jnp.float32) + bf_ref[...]
    m = jnp.max(logits, axis=-1, keepdims=True)
    lse = m + jnp.log(jnp.sum(jnp.exp(logits - m), axis=-1, keepdims=True))
    o_ref[...] = logits - lse


def _final(q, acc, scale2, w2b, b2a, b2b, g2, be2, wf, bf):
    return pl.pallas_call(
        _final_body,
        out_shape=jax.ShapeDtypeStruct((_N, _N_CLASSES), jnp.float32),
    )(q, acc, scale2, w2b, b2a, b2b, g2, be2, wf, bf)


def kernel(x, edge_index, W1a, b1a, W1b, b1b, eps1, g1, be1,
           W2a, b2a, W2b, b2b, eps2, g2, be2, Wf, bf):
    src = edge_index[0].astype(jnp.int32)
    dst2 = jnp.reshape(edge_index[1].astype(jnp.int32),
                       (_E // _CHUNK, _CHUNK))
    scale1 = jnp.reshape(1.0 + eps1, (1, 1))
    scale2 = jnp.reshape(1.0 + eps2, (1, 1))

    p = _project(x, W1a)
    acc1 = _segsum(p, src, dst2)
    q = _mid(p, acc1, scale1, W1b, jnp.reshape(b1a, (1, _DIM)),
             jnp.reshape(b1b, (1, _DIM)), jnp.reshape(g1, (1, _DIM)),
             jnp.reshape(be1, (1, _DIM)), W2a)
    acc2 = _segsum(q, src, dst2)
    return _final(q, acc2, scale2, W2b, jnp.reshape(b2a, (1, _DIM)),
                  jnp.reshape(b2b, (1, _DIM)), jnp.reshape(g2, (1, _DIM)),
                  jnp.reshape(be2, (1, _DIM)), Wf,
                  jnp.reshape(bf, (1, _N_CLASSES)))


# speedup vs baseline: 29.8188x; 1.1746x over previous
import functools

import jax
import jax.numpy as jnp
from jax import lax
from jax.experimental import pallas as pl
from jax.experimental.pallas import tpu as pltpu
from jax.experimental.pallas import tpu_sc as plsc

_N = 10000
_E = 320000
_D_IN = 128
_DIM = 16
_N_CLASSES = 40

_NC = 2
_NS = 16
_NW = _NC * _NS
_EW = _E // _NW
_CHUNK = 2000
_NCHUNK = _EW // _CHUNK
_NPAD = 10240
_ROWS = _NPAD // _NS


def _make_segsum():
    mesh = plsc.VectorSubcoreMesh(core_axis_name="c", subcore_axis_name="s")

    @functools.partial(
        pl.kernel,
        mesh=mesh,
        compiler_params=pltpu.CompilerParams(use_tc_tiling_on_sc=False),
        out_type=jax.ShapeDtypeStruct((_NC, _NPAD, _DIM), jnp.float32),
        scratch_types=[
            pltpu.VMEM((_EW,), jnp.int32),
            pltpu.VMEM((_NCHUNK, _CHUNK), jnp.int32),
            pltpu.VMEM((2, _CHUNK, _DIM), jnp.float32),
            pltpu.VMEM((_ROWS, _DIM), jnp.float32),
            pltpu.VMEM_SHARED((_NPAD, _DIM), jnp.float32),
            pltpu.VMEM_SHARED((_N, _DIM), jnp.float32),
            pltpu.SemaphoreType.DMA,
            pltpu.SemaphoreType.DMA,
            pltpu.SemaphoreType.DMA,
            pltpu.SemaphoreType.DMA,
        ],
    )
    def segsum(p_hbm, src_hbm, dst2_hbm, out_hbm,
               src_v, dst_v, rows_v, zbuf_v, acc_s, p_s,
               sem_s, sem_d, sem_a, sem_b):
        cid = lax.axis_index("c")
        sid = lax.axis_index("s")
        wid = cid * _NS + sid

        ebase = pl.multiple_of(wid * _EW, 8)
        cp_s = pltpu.async_copy(src_hbm.at[pl.ds(ebase, _EW)], src_v, sem_s)
        cp_d = pltpu.async_copy(dst2_hbm.at[pl.ds(wid * _NCHUNK, _NCHUNK)],
                                dst_v, sem_d)

        @pl.when(sid < _NS - 1)
        def _():
            pltpu.sync_copy(p_hbm.at[pl.ds(sid * 640, 640)],
                            p_s.at[pl.ds(sid * 640, 640)])

        @pl.when(sid == _NS - 1)
        def _():
            pltpu.sync_copy(p_hbm.at[pl.ds(9600, _N - 9600)],
                            p_s.at[pl.ds(9600, _N - 9600)])

        def zrow(i, carry):
            zbuf_v[i, :] = jnp.zeros((_DIM,), jnp.float32)
            return carry

        lax.fori_loop(0, _ROWS, zrow, 0)
        pltpu.sync_copy(zbuf_v, acc_s.at[pl.ds(sid * _ROWS, _ROWS)])
        cp_s.wait()
        cp_d.wait()

        plsc.subcore_barrier()
        sems = (sem_a, sem_b)
        gathers = [None, None]
        for i in range(2):
            gathers[i] = pltpu.async_copy(
                p_s.at[src_v.at[pl.ds(i * _CHUNK, _CHUNK)]],
                rows_v.at[i], sems[i])
        for i in range(_NCHUNK):
            b = i % 2
            gathers[b].wait()
            pltpu.sync_copy(rows_v.at[b], acc_s.at[dst_v.at[i]], add=True)
            nxt = i + 2
            if nxt < _NCHUNK:
                gathers[b] = pltpu.async_copy(
                    p_s.at[src_v.at[pl.ds(nxt * _CHUNK, _CHUNK)]],
                    rows_v.at[b], sems[b])
        plsc.subcore_barrier()

        pltpu.sync_copy(acc_s.at[pl.ds(sid * _ROWS, _ROWS)],
                        out_hbm.at[cid, pl.ds(sid * _ROWS, _ROWS)])

    return segsum


_SEGSUM_CACHE = []


def _segsum(p, src, dst2):
    if not _SEGSUM_CACHE:
        _SEGSUM_CACHE.append(_make_segsum())
    return _SEGSUM_CACHE[0](p, src, dst2)


def _proj_body(x_ref, w_ref, o_ref):
    o_ref[...] = jnp.dot(x_ref[...], w_ref[...],
                         preferred_element_type=jnp.float32)


def _project(x, w):
    return pl.pallas_call(
        _proj_body,
        out_shape=jax.ShapeDtypeStruct((_N, _DIM), jnp.float32),
    )(x, w)


_RP = _N // 8
_APAD = _NPAD // 8


def _bn_packed(u, m_ref, g_ref, be_ref):
    mean = jnp.dot(jnp.sum(u, axis=0, keepdims=True), m_ref[...],
                   preferred_element_type=jnp.float32) / _N
    d = u - mean
    var = jnp.dot(jnp.sum(d * d, axis=0, keepdims=True), m_ref[...],
                  preferred_element_type=jnp.float32) / _N
    return g_ref[...] * d * lax.rsqrt(var + 1e-5) + be_ref[...]


def _mid_body(p_ref, acc_ref, sc_ref, w1b_ref, b1a_ref, b1b_ref,
              g1_ref, be1_ref, w2a_ref, m_ref, q_ref):
    u = (sc_ref[0, 0] * p_ref[...] + acc_ref[0, :_RP] + acc_ref[1, :_RP]
         + b1a_ref[...])
    u = jnp.maximum(u, 0.0)
    u = jnp.dot(u, w1b_ref[...], preferred_element_type=jnp.float32)
    u = jnp.maximum(u + b1b_ref[...], 0.0)
    h = _bn_packed(u, m_ref, g1_ref, be1_ref)
    q_ref[...] = jnp.dot(h, w2a_ref[...], preferred_element_type=jnp.float32)


def _mid(p, acc, scale1, w1b_bd, b1a, b1b, g1, be1, w2a_bd, m):
    return pl.pallas_call(
        _mid_body,
        out_shape=jax.ShapeDtypeStruct((_RP, 128), jnp.float32),
    )(p, acc, scale1, w1b_bd, b1a, b1b, g1, be1, w2a_bd, m)


def _bn(u, g_ref, be_ref):
    mean = jnp.mean(u, axis=0, keepdims=True)
    var = jnp.mean((u - mean) ** 2, axis=0, keepdims=True)
    return g_ref[...] * (u - mean) * lax.rsqrt(var + 1e-5) + be_ref[...]


def _final_body(q_ref, acc_ref, sc_ref, w2b_ref, b2a_ref, b2b_ref,
                g2_ref, be2_ref, wf_ref, bf_ref, o_ref):
    u = (sc_ref[0, 0] * q_ref[...] + acc_ref[0, :_N] + acc_ref[1, :_N]
         + b2a_ref[...])
    u = jnp.maximum(u, 0.0)
    u = jnp.dot(u, w2b_ref[...], preferred_element_type=jnp.float32)
    u = jnp.maximum(u + b2b_ref[...], 0.0)
    h = _bn(u, g2_ref, be2_ref)
    logits = jnp.dot(h, wf_ref[...],
                     preferred_element_type=jnp.float32) + bf_ref[...]
    m = jnp.max(logits, axis=-1, keepdims=True)
    lse = m + jnp.log(jnp.sum(jnp.exp(logits - m), axis=-1, keepdims=True))
    o_ref[...] = logits - lse


def _final(q, acc, scale2, w2b, b2a, b2b, g2, be2, wf, bf):
    return pl.pallas_call(
        _final_body,
        out_shape=jax.ShapeDtypeStruct((_N, _N_CLASSES), jnp.float32),
    )(q, acc, scale2, w2b, b2a, b2b, g2, be2, wf, bf)


def kernel(x, edge_index, W1a, b1a, W1b, b1b, eps1, g1, be1,
           W2a, b2a, W2b, b2b, eps2, g2, be2, Wf, bf):
    src = edge_index[0].astype(jnp.int32)
    dst2 = jnp.reshape(edge_index[1].astype(jnp.int32),
                       (_E // _CHUNK, _CHUNK))
    scale1 = jnp.reshape(1.0 + eps1, (1, 1))
    scale2 = jnp.reshape(1.0 + eps2, (1, 1))

    eye8 = jnp.eye(8, dtype=jnp.float32)
    fold = jnp.kron(jnp.ones((8, 8), jnp.float32),
                    jnp.eye(16, dtype=jnp.float32))
    w1b_bd = jnp.kron(eye8, W1b)
    w2a_bd = jnp.kron(eye8, W2a)

    def t8(v):
        return jnp.reshape(jnp.tile(jnp.reshape(v, (1, _DIM)), (8, 1)),
                           (1, 128))

    p = _project(x, W1a)
    acc1 = _segsum(p, src, dst2)
    q = _mid(jnp.reshape(p, (_RP, 128)),
             jnp.reshape(acc1, (_NC, _APAD, 128)), scale1, w1b_bd,
             t8(b1a), t8(b1b), t8(g1), t8(be1), w2a_bd, fold)
    qr = jnp.reshape(q, (_N, _DIM))
    acc2 = _segsum(qr, src, dst2)
    return _final(qr, acc2, scale2, W2b, jnp.reshape(b2a, (1, _DIM)),
                  jnp.reshape(b2b, (1, _DIM)), jnp.reshape(g2, (1, _DIM)),
                  jnp.reshape(be2, (1, _DIM)), Wf,
                  jnp.reshape(bf, (1, _N_CLASSES)))


# speedup vs baseline: 31.6639x; 1.0619x over previous
import functools

import jax
import jax.numpy as jnp
from jax import lax
from jax.experimental import pallas as pl
from jax.experimental.pallas import tpu as pltpu
from jax.experimental.pallas import tpu_sc as plsc

_N = 10000
_E = 320000
_D_IN = 128
_DIM = 16
_N_CLASSES = 40

_NC = 2
_NS = 16
_NW = _NC * _NS
_EW = _E // _NW
_CHUNK = 2000
_NCHUNK = _EW // _CHUNK
_NPAD = 10240
_ROWS = _NPAD // _NS


def _make_segsum():
    mesh = plsc.VectorSubcoreMesh(core_axis_name="c", subcore_axis_name="s")

    @functools.partial(
        pl.kernel,
        mesh=mesh,
        compiler_params=pltpu.CompilerParams(use_tc_tiling_on_sc=False),
        out_type=jax.ShapeDtypeStruct((_NC, _NPAD, _DIM), jnp.float32),
        scratch_types=[
            pltpu.VMEM((_EW,), jnp.int32),
            pltpu.VMEM((_NCHUNK, _CHUNK), jnp.int32),
            pltpu.VMEM((2, _CHUNK, _DIM), jnp.float32),
            pltpu.VMEM((_ROWS, _DIM), jnp.float32),
            pltpu.VMEM_SHARED((_NPAD, _DIM), jnp.float32),
            pltpu.VMEM_SHARED((_N, _DIM), jnp.float32),
            pltpu.SemaphoreType.DMA,
            pltpu.SemaphoreType.DMA,
            pltpu.SemaphoreType.DMA,
            pltpu.SemaphoreType.DMA,
        ],
    )
    def segsum(p_hbm, src_hbm, dst2_hbm, out_hbm,
               src_v, dst_v, rows_v, zbuf_v, acc_s, p_s,
               sem_s, sem_d, sem_a, sem_b):
        cid = lax.axis_index("c")
        sid = lax.axis_index("s")
        wid = cid * _NS + sid

        ebase = pl.multiple_of(wid * _EW, 8)
        cp_s = pltpu.async_copy(src_hbm.at[pl.ds(ebase, _EW)], src_v, sem_s)
        cp_d = pltpu.async_copy(dst2_hbm.at[pl.ds(wid * _NCHUNK, _NCHUNK)],
                                dst_v, sem_d)

        @pl.when(sid < _NS - 1)
        def _():
            pltpu.sync_copy(p_hbm.at[pl.ds(sid * 640, 640)],
                            p_s.at[pl.ds(sid * 640, 640)])

        @pl.when(sid == _NS - 1)
        def _():
            pltpu.sync_copy(p_hbm.at[pl.ds(9600, _N - 9600)],
                            p_s.at[pl.ds(9600, _N - 9600)])

        def zrow(i, carry):
            zbuf_v[i, :] = jnp.zeros((_DIM,), jnp.float32)
            return carry

        lax.fori_loop(0, _ROWS, zrow, 0)
        pltpu.sync_copy(zbuf_v, acc_s.at[pl.ds(sid * _ROWS, _ROWS)])
        cp_s.wait()
        cp_d.wait()

        plsc.subcore_barrier()
        sems = (sem_a, sem_b)
        gathers = [None, None]
        for i in range(2):
            gathers[i] = pltpu.async_copy(
                p_s.at[src_v.at[pl.ds(i * _CHUNK, _CHUNK)]],
                rows_v.at[i], sems[i])
        for i in range(_NCHUNK):
            b = i % 2
            gathers[b].wait()
            pltpu.sync_copy(rows_v.at[b], acc_s.at[dst_v.at[i]], add=True)
            nxt = i + 2
            if nxt < _NCHUNK:
                gathers[b] = pltpu.async_copy(
                    p_s.at[src_v.at[pl.ds(nxt * _CHUNK, _CHUNK)]],
                    rows_v.at[b], sems[b])
        plsc.subcore_barrier()

        pltpu.sync_copy(acc_s.at[pl.ds(sid * _ROWS, _ROWS)],
                        out_hbm.at[cid, pl.ds(sid * _ROWS, _ROWS)])

    return segsum


_SEGSUM_CACHE = []


def _segsum(p, src, dst2):
    if not _SEGSUM_CACHE:
        _SEGSUM_CACHE.append(_make_segsum())
    return _SEGSUM_CACHE[0](p, src, dst2)


def _proj_body(x_ref, w_ref, o_ref):
    o_ref[...] = jnp.dot(x_ref[...], w_ref[...],
                         preferred_element_type=jnp.float32)


def _project(x, w):
    return pl.pallas_call(
        _proj_body,
        out_shape=jax.ShapeDtypeStruct((_N, _DIM), jnp.float32),
    )(x, w)


_RP = _N // 8
_APAD = _NPAD // 8


def _bn_packed(u, m_ref, g_ref, be_ref):
    mean = jnp.dot(jnp.sum(u, axis=0, keepdims=True), m_ref[...],
                   preferred_element_type=jnp.float32) / _N
    d = u - mean
    var = jnp.dot(jnp.sum(d * d, axis=0, keepdims=True), m_ref[...],
                  preferred_element_type=jnp.float32) / _N
    return g_ref[...] * d * lax.rsqrt(var + 1e-5) + be_ref[...]


def _mid_body(p_ref, acc_ref, sc_ref, w1b_ref, b1a_ref, b1b_ref,
              g1_ref, be1_ref, w2a_ref, m_ref, q_ref):
    u = (sc_ref[0, 0] * p_ref[...] + acc_ref[0, :_RP] + acc_ref[1, :_RP]
         + b1a_ref[...])
    u = jnp.maximum(u, 0.0)
    u = jnp.dot(u, w1b_ref[...], preferred_element_type=jnp.float32)
    u = jnp.maximum(u + b1b_ref[...], 0.0)
    h = _bn_packed(u, m_ref, g1_ref, be1_ref)
    q_ref[...] = jnp.dot(h, w2a_ref[...], preferred_element_type=jnp.float32)


def _mid(p, acc, scale1, w1b_bd, b1a, b1b, g1, be1, w2a_bd, m):
    return pl.pallas_call(
        _mid_body,
        out_shape=jax.ShapeDtypeStruct((_RP, 128), jnp.float32),
    )(p, acc, scale1, w1b_bd, b1a, b1b, g1, be1, w2a_bd, m)


def _final_body(q_ref, acc_ref, sc_ref, w2b_ref, b2a_ref, b2b_ref,
                g2_ref, be2_ref, m_ref, wf_ref, bf_ref, o_ref):
    u = (sc_ref[0, 0] * q_ref[...] + acc_ref[0, :_RP] + acc_ref[1, :_RP]
         + b2a_ref[...])
    u = jnp.maximum(u, 0.0)
    u = jnp.dot(u, w2b_ref[...], preferred_element_type=jnp.float32)
    u = jnp.maximum(u + b2b_ref[...], 0.0)
    h = _bn_packed(u, m_ref, g2_ref, be2_ref)
    logits = jnp.dot(h, wf_ref[...],
                     preferred_element_type=jnp.float32) + bf_ref[...]
    for g in range(8):
        l = logits[:, g * _N_CLASSES:(g + 1) * _N_CLASSES]
        m = jnp.max(l, axis=-1, keepdims=True)
        lse = m + jnp.log(jnp.sum(jnp.exp(l - m), axis=-1, keepdims=True))
        o_ref[:, g * _N_CLASSES:(g + 1) * _N_CLASSES] = l - lse


def _final(q, acc, scale2, w2b_bd, b2a, b2b, g2, be2, m, wf_bd, bf):
    return pl.pallas_call(
        _final_body,
        out_shape=jax.ShapeDtypeStruct((_RP, 8 * _N_CLASSES), jnp.float32),
    )(q, acc, scale2, w2b_bd, b2a, b2b, g2, be2, m, wf_bd, bf)


def kernel(x, edge_index, W1a, b1a, W1b, b1b, eps1, g1, be1,
           W2a, b2a, W2b, b2b, eps2, g2, be2, Wf, bf):
    src = edge_index[0].astype(jnp.int32)
    dst2 = jnp.reshape(edge_index[1].astype(jnp.int32),
                       (_E // _CHUNK, _CHUNK))
    scale1 = jnp.reshape(1.0 + eps1, (1, 1))
    scale2 = jnp.reshape(1.0 + eps2, (1, 1))

    eye8 = jnp.eye(8, dtype=jnp.float32)
    fold = jnp.kron(jnp.ones((8, 8), jnp.float32),
                    jnp.eye(16, dtype=jnp.float32))
    w1b_bd = jnp.kron(eye8, W1b)
    w2a_bd = jnp.kron(eye8, W2a)
    w2b_bd = jnp.kron(eye8, W2b)
    wf_bd = jnp.kron(eye8, Wf)

    def t8(v):
        n = v.shape[-1]
        return jnp.reshape(jnp.tile(jnp.reshape(v, (1, n)), (8, 1)),
                           (1, 8 * n))

    p = _project(x, W1a)
    acc1 = _segsum(p, src, dst2)
    q = _mid(jnp.reshape(p, (_RP, 128)),
             jnp.reshape(acc1, (_NC, _APAD, 128)), scale1, w1b_bd,
             t8(b1a), t8(b1b), t8(g1), t8(be1), w2a_bd, fold)
    acc2 = _segsum(jnp.reshape(q, (_N, _DIM)), src, dst2)
    out = _final(q, jnp.reshape(acc2, (_NC, _APAD, 128)), scale2, w2b_bd,
                 t8(b2a), t8(b2b), t8(g2), t8(be2), fold, wf_bd, t8(bf))
    return jnp.reshape(out, (_N, _N_CLASSES))


# speedup vs baseline: 33.8310x; 1.0684x over previous
import functools

import jax
import jax.numpy as jnp
from jax import lax
from jax.experimental import pallas as pl
from jax.experimental.pallas import tpu as pltpu
from jax.experimental.pallas import tpu_sc as plsc

_N = 10000
_E = 320000
_D_IN = 128
_DIM = 16
_N_CLASSES = 40

_NC = 2
_NS = 16
_NW = _NC * _NS
_EW = _E // _NW
_CHUNK = 2000
_NCHUNK = _EW // _CHUNK
_NPAD = 10240
_ROWS = _NPAD // _NS


def _make_segsum():
    mesh = plsc.VectorSubcoreMesh(core_axis_name="c", subcore_axis_name="s")

    @functools.partial(
        pl.kernel,
        mesh=mesh,
        compiler_params=pltpu.CompilerParams(use_tc_tiling_on_sc=False),
        out_type=jax.ShapeDtypeStruct((_NC, _NPAD, _DIM), jnp.float32),
        scratch_types=[
            pltpu.VMEM((_EW,), jnp.int32),
            pltpu.VMEM((_NCHUNK, _CHUNK), jnp.int32),
            pltpu.VMEM((2, _CHUNK, _DIM), jnp.float32),
            pltpu.VMEM((_ROWS, _DIM), jnp.float32),
            pltpu.VMEM_SHARED((_NPAD, _DIM), jnp.float32),
            pltpu.VMEM_SHARED((_N, _DIM), jnp.float32),
            pltpu.SemaphoreType.DMA,
            pltpu.SemaphoreType.DMA,
            pltpu.SemaphoreType.DMA,
            pltpu.SemaphoreType.DMA,
        ],
    )
    def segsum(p_hbm, src_hbm, dst2_hbm, out_hbm,
               src_v, dst_v, rows_v, zbuf_v, acc_s, p_s,
               sem_s, sem_d, sem_a, sem_b):
        cid = lax.axis_index("c")
        sid = lax.axis_index("s")
        wid = cid * _NS + sid

        ebase = pl.multiple_of(wid * _EW, 8)
        cp_s = pltpu.async_copy(src_hbm.at[pl.ds(ebase, _EW)], src_v, sem_s)
        cp_d = pltpu.async_copy(dst2_hbm.at[pl.ds(wid * _NCHUNK, _NCHUNK)],
                                dst_v, sem_d)

        @pl.when(sid < _NS - 1)
        def _():
            pltpu.sync_copy(p_hbm.at[pl.ds(sid * 640, 640)],
                            p_s.at[pl.ds(sid * 640, 640)])

        @pl.when(sid == _NS - 1)
        def _():
            pltpu.sync_copy(p_hbm.at[pl.ds(9600, _N - 9600)],
                            p_s.at[pl.ds(9600, _N - 9600)])

        def zrow(i, carry):
            zbuf_v[i, :] = jnp.zeros((_DIM,), jnp.float32)
            return carry

        lax.fori_loop(0, _ROWS, zrow, 0)
        pltpu.sync_copy(zbuf_v, acc_s.at[pl.ds(sid * _ROWS, _ROWS)])
        cp_s.wait()
        cp_d.wait()

        plsc.subcore_barrier()
        sems = (sem_a, sem_b)
        gathers = [None, None]
        for i in range(2):
            gathers[i] = pltpu.async_copy(
                p_s.at[src_v.at[pl.ds(i * _CHUNK, _CHUNK)]],
                rows_v.at[i], sems[i])
        for i in range(_NCHUNK):
            b = i % 2
            gathers[b].wait()
            pltpu.sync_copy(rows_v.at[b], acc_s.at[dst_v.at[i]], add=True)
            nxt = i + 2
            if nxt < _NCHUNK:
                gathers[b] = pltpu.async_copy(
                    p_s.at[src_v.at[pl.ds(nxt * _CHUNK, _CHUNK)]],
                    rows_v.at[b], sems[b])
        plsc.subcore_barrier()

        pltpu.sync_copy(acc_s.at[pl.ds(sid * _ROWS, _ROWS)],
                        out_hbm.at[cid, pl.ds(sid * _ROWS, _ROWS)])

    return segsum


_SEGSUM_CACHE = []


def _segsum(p, src, dst2):
    if not _SEGSUM_CACHE:
        _SEGSUM_CACHE.append(_make_segsum())
    return _SEGSUM_CACHE[0](p, src, dst2)


def _proj_body(x_ref, w_ref, o_ref):
    o_ref[...] = jnp.dot(x_ref[...], w_ref[...],
                         preferred_element_type=jnp.float32)


def _project(x, w):
    return pl.pallas_call(
        _proj_body,
        out_shape=jax.ShapeDtypeStruct((_N, _DIM), jnp.float32),
    )(x, w)


_RP = _N // 8
_APAD = _NPAD // 8


def _bn_packed(u, m_ref, g_ref, be_ref):
    mean = jnp.dot(jnp.sum(u, axis=0, keepdims=True), m_ref[...],
                   preferred_element_type=jnp.float32) / _N
    d = u - mean
    var = jnp.dot(jnp.sum(d * d, axis=0, keepdims=True), m_ref[...],
                  preferred_element_type=jnp.float32) / _N
    return g_ref[...] * d * lax.rsqrt(var + 1e-5) + be_ref[...]


def _mid_body(p_ref, acc_ref, sc_ref, w1b_ref, b1a_ref, b1b_ref,
              g1_ref, be1_ref, w2a_ref, m_ref, q_ref):
    u = (sc_ref[0, 0] * p_ref[...] + acc_ref[0, :_RP] + acc_ref[1, :_RP]
         + b1a_ref[...])
    u = jnp.maximum(u, 0.0)
    u = jnp.dot(u, w1b_ref[...], preferred_element_type=jnp.float32)
    u = jnp.maximum(u + b1b_ref[...], 0.0)
    h = _bn_packed(u, m_ref, g1_ref, be1_ref)
    q_ref[...] = jnp.dot(h, w2a_ref[...], preferred_element_type=jnp.float32)


def _mid(p, acc, scale1, w1b_bd, b1a, b1b, g1, be1, w2a_bd, m):
    return pl.pallas_call(
        _mid_body,
        out_shape=jax.ShapeDtypeStruct((_RP, 128), jnp.float32),
    )(p, acc, scale1, w1b_bd, b1a, b1b, g1, be1, w2a_bd, m)


def _final_body(q_ref, acc_ref, sc_ref, w2b_ref, b2a_ref, b2b_ref,
                g2_ref, be2_ref, m_ref, wf_ref, bf_ref, sumk_ref, o_ref):
    u = (sc_ref[0, 0] * q_ref[...] + acc_ref[0, :_RP] + acc_ref[1, :_RP]
         + b2a_ref[...])
    u = jnp.maximum(u, 0.0)
    u = jnp.dot(u, w2b_ref[...], preferred_element_type=jnp.float32)
    u = jnp.maximum(u + b2b_ref[...], 0.0)
    h = _bn_packed(u, m_ref, g2_ref, be2_ref)
    logits = jnp.dot(h, wf_ref[...],
                     preferred_element_type=jnp.float32) + bf_ref[...]
    mrow = jnp.max(logits, axis=-1, keepdims=True)
    e = jnp.exp(logits - mrow)
    s = jnp.dot(e, sumk_ref[...], preferred_element_type=jnp.float32)
    o_ref[...] = logits - mrow - jnp.log(s)


def _final(q, acc, scale2, w2b_bd, b2a, b2b, g2, be2, m, wf_bd, bf, sumk):
    return pl.pallas_call(
        _final_body,
        out_shape=jax.ShapeDtypeStruct((_RP, 8 * _N_CLASSES), jnp.float32),
    )(q, acc, scale2, w2b_bd, b2a, b2b, g2, be2, m, wf_bd, bf, sumk)


def kernel(x, edge_index, W1a, b1a, W1b, b1b, eps1, g1, be1,
           W2a, b2a, W2b, b2b, eps2, g2, be2, Wf, bf):
    src = edge_index[0].astype(jnp.int32)
    dst2 = jnp.reshape(edge_index[1].astype(jnp.int32),
                       (_E // _CHUNK, _CHUNK))
    scale1 = jnp.reshape(1.0 + eps1, (1, 1))
    scale2 = jnp.reshape(1.0 + eps2, (1, 1))

    eye8 = jnp.eye(8, dtype=jnp.float32)
    fold = jnp.kron(jnp.ones((8, 8), jnp.float32),
                    jnp.eye(16, dtype=jnp.float32))
    w1b_bd = jnp.kron(eye8, W1b)
    w2a_bd = jnp.kron(eye8, W2a)
    w2b_bd = jnp.kron(eye8, W2b)
    wf_bd = jnp.kron(eye8, Wf)

    def t8(v):
        n = v.shape[-1]
        return jnp.reshape(jnp.tile(jnp.reshape(v, (1, n)), (8, 1)),
                           (1, 8 * n))

    p = _project(x, W1a)
    acc1 = _segsum(p, src, dst2)
    q = _mid(jnp.reshape(p, (_RP, 128)),
             jnp.reshape(acc1, (_NC, _APAD, 128)), scale1, w1b_bd,
             t8(b1a), t8(b1b), t8(g1), t8(be1), w2a_bd, fold)
    acc2 = _segsum(jnp.reshape(q, (_N, _DIM)), src, dst2)
    sumk = jnp.kron(eye8, jnp.ones((_N_CLASSES, _N_CLASSES), jnp.float32))
    out = _final(q, jnp.reshape(acc2, (_NC, _APAD, 128)), scale2, w2b_bd,
                 t8(b2a), t8(b2b), t8(g2), t8(be2), fold, wf_bd, t8(bf),
                 sumk)
    return jnp.reshape(out, (_N, _N_CLASSES))
